# baseline (speedup 1.0000x reference)
import jax
import jax.numpy as jnp
from jax import lax
from jax.experimental import pallas as pl
from jax.experimental.pallas import tpu as pltpu
from jax.experimental.pallas import tpu_sc as plsc

BATCH = 16384
EDIM = 64
PDIM = 2 * EDIM
ANUM = 32
NUM_CORES = 2
NUM_SUBCORES = 16
NW = NUM_CORES * NUM_SUBCORES
BPW = BATCH // NW
CHUNK = 128
NCH = BPW // CHUNK
VREG = 16


def _sc_gather_body(uid_hbm, iid_hbm, e1_hbm, e2_hbm,
                    u1_out, u2_out, i1_out, i2_out,
                    uidx_v, iidx_v, bufa0, bufa1, bufb0, bufb1,
                    sema0, sema1, semb0, semb1):
    c = lax.axis_index("c")
    s = lax.axis_index("s")
    wid = s * NUM_CORES + c
    base = wid * BPW
    pltpu.sync_copy(uid_hbm.at[pl.ds(base, BPW)], uidx_v)
    pltpu.sync_copy(iid_hbm.at[pl.ds(base, BPW)], iidx_v)
    for idx_v in (uidx_v, iidx_v):
        for j in range(BPW // VREG):
            sl = pl.ds(j * VREG, VREG)
            idx_v[sl] = lax.shift_right_logical(idx_v[sl], 1)

    jobs = []
    for idx_v, out1, out2 in ((uidx_v, u1_out, u2_out),
                              (iidx_v, i1_out, i2_out)):
        for ch in range(NCH):
            jobs.append((idx_v, out1, out2, ch * CHUNK))
    bufsa = (bufa0, bufa1)
    bufsb = (bufb0, bufb1)
    semsa = (sema0, sema1)
    semsb = (semb0, semb1)
    copies = [None, None]

    def fire(k):
        idx_v, _, _, off = jobs[k]
        nb = k % 2
        idx = idx_v.at[pl.ds(off, CHUNK)]
        copies[nb] = (
            pltpu.async_copy(e1_hbm.at[idx], bufsa[nb], semsa[nb]),
            pltpu.async_copy(e2_hbm.at[idx], bufsb[nb], semsb[nb]),
        )

    fire(0)
    for k in range(len(jobs)):
        if k + 1 < len(jobs):
            fire(k + 1)
        _, out1, out2, off = jobs[k]
        ca, cb = copies[k % 2]
        ca.wait()
        cb.wait()
        pltpu.sync_copy(bufsa[k % 2], out1.at[pl.ds(base + off, CHUNK)])
        pltpu.sync_copy(bufsb[k % 2], out2.at[pl.ds(base + off, CHUNK)])


def _sc_gather(user_id, item_id, e1v, e2v):
    mesh = plsc.VectorSubcoreMesh(core_axis_name="c", subcore_axis_name="s")
    rows = jax.ShapeDtypeStruct((BATCH, PDIM), jnp.float32)
    fn = pl.kernel(
        _sc_gather_body,
        out_type=(rows, rows, rows, rows),
        mesh=mesh,
        scratch_types=(
            pltpu.VMEM((BPW,), jnp.int32),
            pltpu.VMEM((BPW,), jnp.int32),
            pltpu.VMEM((CHUNK, PDIM), jnp.float32),
            pltpu.VMEM((CHUNK, PDIM), jnp.float32),
            pltpu.VMEM((CHUNK, PDIM), jnp.float32),
            pltpu.VMEM((CHUNK, PDIM), jnp.float32),
            pltpu.SemaphoreType.DMA,
            pltpu.SemaphoreType.DMA,
            pltpu.SemaphoreType.DMA,
            pltpu.SemaphoreType.DMA,
        ),
    )
    return fn(user_id, item_id, e1v, e2v)


def _tc_body(uid_ref, iid_ref, u1p_ref, u2p_ref, i1p_ref, i2p_ref,
             w_ref, b_ref, prob_ref, pu_ref, pi_ref):
    um = (uid_ref[...] & 1) == 1
    im = (iid_ref[...] & 1) == 1
    u1p, u2p = u1p_ref[...], u2p_ref[...]
    i1p, i2p = i1p_ref[...], i2p_ref[...]
    u1 = jnp.where(um, u1p[:, EDIM:], u1p[:, :EDIM])
    u2 = jnp.where(um, u2p[:, EDIM:], u2p[:, :EDIM])
    i1 = jnp.where(im, i1p[:, EDIM:], i1p[:, :EDIM])
    i2 = jnp.where(im, i2p[:, EDIM:], i2p[:, :EDIM])
    w = w_ref[...]
    b = b_ref[...]
    dn = (((1,), (1,)), ((), ()))
    pu_ref[...] = lax.dot_general(u1, w, dn,
                                  preferred_element_type=jnp.float32) + b
    pi_ref[...] = lax.dot_general(i1, w, dn,
                                  preferred_element_type=jnp.float32) + b
    d = jnp.sum(u1 * i1 + u2 * i2, axis=1, keepdims=True)
    prob_ref[...] = jax.nn.sigmoid(d)


def _tc_compute(user_id, item_id, u1p, u2p, i1p, i2p, w, b):
    blk = 2048
    row_spec = pl.BlockSpec((blk, PDIM), lambda j: (j, 0))
    id_spec = pl.BlockSpec((blk, 1), lambda j: (j, 0))
    prob, pu, pi = pl.pallas_call(
        _tc_body,
        grid=(BATCH // blk,),
        in_specs=[
            id_spec, id_spec,
            row_spec, row_spec, row_spec, row_spec,
            pl.BlockSpec((ANUM, EDIM), lambda j: (0, 0)),
            pl.BlockSpec((1, ANUM), lambda j: (0, 0)),
        ],
        out_specs=[
            pl.BlockSpec((blk, 1), lambda j: (j, 0)),
            pl.BlockSpec((blk, ANUM), lambda j: (j, 0)),
            pl.BlockSpec((blk, ANUM), lambda j: (j, 0)),
        ],
        out_shape=[
            jax.ShapeDtypeStruct((BATCH, 1), jnp.float32),
            jax.ShapeDtypeStruct((BATCH, ANUM), jnp.float32),
            jax.ShapeDtypeStruct((BATCH, ANUM), jnp.float32),
        ],
    )(user_id.reshape(BATCH, 1), item_id.reshape(BATCH, 1),
      u1p, u2p, i1p, i2p, w, b.reshape(1, ANUM))
    return prob.reshape(BATCH), pu, pi


def kernel(user_id, item_id, Eu1, Eu2, Ei1, Ei2, W, b):
    e1v = Eu1.reshape(Eu1.shape[0] // 2, PDIM)
    e2v = Eu2.reshape(Eu2.shape[0] // 2, PDIM)
    u1p, u2p, i1p, i2p = _sc_gather(user_id, item_id, e1v, e2v)
    return _tc_compute(user_id, item_id, u1p, u2p, i1p, i2p, W, b)


# speedup vs baseline: 1.3024x; 1.3024x over previous
import jax
import jax.numpy as jnp
from jax import lax
from jax.experimental import pallas as pl
from jax.experimental.pallas import tpu as pltpu
from jax.experimental.pallas import tpu_sc as plsc

BATCH = 16384
EDIM = 64
PDIM = 2 * EDIM
ANUM = 32
NUM_CORES = 2
NUM_SUBCORES = 16
NW = NUM_CORES * NUM_SUBCORES
BPW = BATCH // NW
CHUNK = 256
NCH = BPW // CHUNK


def _sc_gather_body(uid_hbm, iid_hbm, c_hbm, u_out, i_out,
                    uidx_v, iidx_v, buf0, buf1, sem0, sem1):
    c = lax.axis_index("c")
    s = lax.axis_index("s")
    wid = s * NUM_CORES + c
    base = wid * BPW
    pltpu.sync_copy(uid_hbm.at[pl.ds(base, BPW)], uidx_v)
    pltpu.sync_copy(iid_hbm.at[pl.ds(base, BPW)], iidx_v)

    jobs = []
    for idx_v, out in ((uidx_v, u_out), (iidx_v, i_out)):
        for ch in range(NCH):
            jobs.append((idx_v, out, ch * CHUNK))
    bufs = (buf0, buf1)
    sems = (sem0, sem1)
    copies = [None, None]

    def fire(k):
        idx_v, _, off = jobs[k]
        nb = k % 2
        copies[nb] = pltpu.async_copy(
            c_hbm.at[idx_v.at[pl.ds(off, CHUNK)]], bufs[nb], sems[nb])

    fire(0)
    for k in range(len(jobs)):
        if k + 1 < len(jobs):
            fire(k + 1)
        _, out, off = jobs[k]
        copies[k % 2].wait()
        pltpu.sync_copy(bufs[k % 2], out.at[pl.ds(base + off, CHUNK)])


def _sc_gather(user_id, item_id, c_tab):
    mesh = plsc.VectorSubcoreMesh(core_axis_name="c", subcore_axis_name="s")
    rows = jax.ShapeDtypeStruct((BATCH, PDIM), jnp.float32)
    fn = pl.kernel(
        _sc_gather_body,
        out_type=(rows, rows),
        mesh=mesh,
        scratch_types=(
            pltpu.VMEM((BPW,), jnp.int32),
            pltpu.VMEM((BPW,), jnp.int32),
            pltpu.VMEM((CHUNK, PDIM), jnp.float32),
            pltpu.VMEM((CHUNK, PDIM), jnp.float32),
            pltpu.SemaphoreType.DMA,
            pltpu.SemaphoreType.DMA,
        ),
    )
    return fn(user_id, item_id, c_tab)


def _tc_body(u_ref, i_ref, w_ref, b_ref, prob_ref, pu_ref, pi_ref):
    u = u_ref[...]
    i = i_ref[...]
    w = w_ref[...]
    b = b_ref[...]
    dn = (((1,), (1,)), ((), ()))
    pu_ref[...] = lax.dot_general(u[:, :EDIM], w, dn,
                                  preferred_element_type=jnp.float32) + b
    pi_ref[...] = lax.dot_general(i[:, :EDIM], w, dn,
                                  preferred_element_type=jnp.float32) + b
    d = jnp.sum(u * i, axis=1, keepdims=True)
    prob_ref[...] = jax.nn.sigmoid(d)


def _tc_compute(u, i, w, b):
    blk = 2048
    row_spec = pl.BlockSpec((blk, PDIM), lambda j: (j, 0))
    prob, pu, pi = pl.pallas_call(
        _tc_body,
        grid=(BATCH // blk,),
        in_specs=[
            row_spec, row_spec,
            pl.BlockSpec((ANUM, EDIM), lambda j: (0, 0)),
            pl.BlockSpec((1, ANUM), lambda j: (0, 0)),
        ],
        out_specs=[
            pl.BlockSpec((blk, 1), lambda j: (j, 0)),
            pl.BlockSpec((blk, ANUM), lambda j: (j, 0)),
            pl.BlockSpec((blk, ANUM), lambda j: (j, 0)),
        ],
        out_shape=[
            jax.ShapeDtypeStruct((BATCH, 1), jnp.float32),
            jax.ShapeDtypeStruct((BATCH, ANUM), jnp.float32),
            jax.ShapeDtypeStruct((BATCH, ANUM), jnp.float32),
        ],
    )(u, i, w, b.reshape(1, ANUM))
    return prob.reshape(BATCH), pu, pi


def kernel(user_id, item_id, Eu1, Eu2, Ei1, Ei2, W, b):
    c_tab = jnp.concatenate([Eu1, Eu2], axis=1)
    u, i = _sc_gather(user_id, item_id, c_tab)
    return _tc_compute(u, i, W, b)
